# baseline (speedup 1.0000x reference)
import functools

import jax
import jax.numpy as jnp
from jax import lax
from jax.experimental import pallas as pl
from jax.experimental.pallas import tpu as pltpu
from jax.experimental.pallas import tpu_sc as plsc

E = 8
H = 1024
T = 4096
TOPK = 2
BM = 512
CAP = T * TOPK + E * BM
NB = CAP // BM

NC, NS, NL = 2, 16, 16
NW = NC * NS
TPW = T // NW
DCH = 64
CCH = 16


@functools.cache
def _mesh():
    return plsc.VectorSubcoreMesh(core_axis_name="c", subcore_axis_name="s")


def _router_body(x_ref, gw_ref, oh1_ref, oh2_ref, wa_ref, wb_ref, xb_ref):
    x = x_ref[...]
    logits = lax.dot_general(
        x, gw_ref[...], (((1,), (1,)), ((), ())),
        precision=lax.Precision.DEFAULT,
        preferred_element_type=jnp.float32)
    m = jnp.max(logits, axis=-1, keepdims=True)
    el = jnp.exp(logits - m)
    probs = el / jnp.sum(el, axis=-1, keepdims=True)

    eidx = lax.broadcasted_iota(jnp.int32, probs.shape, 1)
    m1 = jnp.max(probs, axis=-1, keepdims=True)
    i1 = jnp.min(jnp.where(probs == m1, eidx, E), axis=-1, keepdims=True)
    oh1 = (eidx == i1).astype(jnp.float32)
    probs2 = jnp.where(oh1 > 0, -1.0, probs)
    m2 = jnp.max(probs2, axis=-1, keepdims=True)
    i2 = jnp.min(jnp.where(probs2 == m2, eidx, E), axis=-1, keepdims=True)
    oh2 = (eidx == i2).astype(jnp.float32)

    oh1_ref[...] = oh1
    oh2_ref[...] = oh2
    wa_ref[...] = jnp.broadcast_to(m1, (x.shape[0], NL))
    wb_ref[...] = jnp.broadcast_to(m2, (x.shape[0], NL))
    xb_ref[...] = x.astype(jnp.bfloat16)


def _router(x, gate_w):
    return pl.pallas_call(
        _router_body,
        in_specs=[
            pl.BlockSpec((T, H), lambda: (0, 0)),
            pl.BlockSpec((E, H), lambda: (0, 0)),
        ],
        out_specs=[
            pl.BlockSpec((T, E), lambda: (0, 0)),
            pl.BlockSpec((T, E), lambda: (0, 0)),
            pl.BlockSpec((T, NL), lambda: (0, 0)),
            pl.BlockSpec((T, NL), lambda: (0, 0)),
            pl.BlockSpec((T, H), lambda: (0, 0)),
        ],
        out_shape=[
            jax.ShapeDtypeStruct((T, E), jnp.float32),
            jax.ShapeDtypeStruct((T, E), jnp.float32),
            jax.ShapeDtypeStruct((T, NL), jnp.float32),
            jax.ShapeDtypeStruct((T, NL), jnp.float32),
            jax.ShapeDtypeStruct((T, H), jnp.bfloat16),
        ],
    )(x, gate_w)


def _positions(oh1, oh2):
    onehot = jnp.concatenate([oh1, oh2], axis=0)
    counts = jnp.sum(onehot, axis=0)
    csum = jnp.cumsum(onehot, axis=0)
    rank = jnp.sum(csum * onehot, axis=1) - 1.0
    pc = jnp.ceil(counts / BM) * BM
    po = jnp.concatenate([jnp.zeros((1,), jnp.float32),
                          jnp.cumsum(pc)[:-1]])
    pos = (jnp.sum(onehot * po[None, :], axis=1) + rank).astype(jnp.int32)
    pa, pb = pos[:T], pos[T:]

    bstart = (jnp.arange(NB, dtype=jnp.float32) * BM)
    act = (bstart[:, None] >= po[None, :]) & (bstart[:, None] < (po + pc)[None, :])
    bexp = jnp.argmax(act, axis=1).astype(jnp.int32)
    bact = jnp.any(act, axis=1).astype(jnp.int32)
    return pa, pb, bexp, bact


H2 = H // 2


@functools.cache
def _dispatch_kernel():
    return pl.kernel(
        _dispatch_body,
        out_type=jax.ShapeDtypeStruct((CAP, H2), jnp.int32),
        mesh=_mesh(),
        scratch_types=[
            pltpu.VMEM((DCH, H2), jnp.int32),
            pltpu.VMEM((DCH,), jnp.int32),
            pltpu.VMEM((DCH,), jnp.int32),
            pltpu.SemaphoreType.DMA,
            pltpu.SemaphoreType.DMA,
        ],
    )


def _dispatch_body(x_hbm, pa_hbm, pb_hbm, xs_hbm, xbuf, ia, ib, s1, s2):
    wid = lax.axis_index("s") * NC + lax.axis_index("c")
    for c in range(TPW // DCH):
        base = wid * TPW + c * DCH
        pltpu.sync_copy(x_hbm.at[pl.ds(base, DCH)], xbuf)
        pltpu.sync_copy(pa_hbm.at[pl.ds(base, DCH)], ia)
        pltpu.sync_copy(pb_hbm.at[pl.ds(base, DCH)], ib)
        ca = pltpu.async_copy(xbuf, xs_hbm.at[ia], s1)
        cb = pltpu.async_copy(xbuf, xs_hbm.at[ib], s2)
        ca.wait()
        cb.wait()


def _mm_body(be_ref, ba_ref, xs_ref, w_ref, o_ref):
    b = pl.program_id(0)

    @pl.when(ba_ref[b] == 1)
    def _():
        w = w_ref[pl.ds(be_ref[b], 1)]
        o_ref[...] = lax.dot_general(
            xs_ref[...], w[0].astype(jnp.bfloat16), (((1,), (1,)), ((), ())),
            precision=lax.Precision.DEFAULT,
            preferred_element_type=jnp.float32)


def _grouped_matmul(bexp, bact, xs, expert_w):
    grid_spec = pltpu.PrefetchScalarGridSpec(
        num_scalar_prefetch=2,
        grid=(NB,),
        in_specs=[
            pl.BlockSpec((BM, H),
                         lambda b, be, ba: (jnp.where(ba[b] == 1, b, 0), 0)),
            pl.BlockSpec((E, H, H), lambda b, be, ba: (0, 0, 0)),
        ],
        out_specs=pl.BlockSpec(
            (BM, H), lambda b, be, ba: (jnp.where(ba[b] == 1, b, NB), 0)),
    )
    return pl.pallas_call(
        _mm_body,
        grid_spec=grid_spec,
        out_shape=jax.ShapeDtypeStruct((CAP + BM, H), jnp.float32),
    )(bexp, bact, xs, expert_w)


@functools.cache
def _combine_kernel():
    return pl.kernel(
        _combine_body,
        out_type=jax.ShapeDtypeStruct((T, H), jnp.float32),
        mesh=_mesh(),
        scratch_types=[
            pltpu.VMEM((CCH, H), jnp.float32),
            pltpu.VMEM((CCH, H), jnp.float32),
            pltpu.VMEM((CCH, H), jnp.float32),
            pltpu.VMEM((CCH, H), jnp.float32),
            pltpu.VMEM((CCH, H), jnp.float32),
            pltpu.VMEM((TPW,), jnp.int32),
            pltpu.VMEM((TPW,), jnp.int32),
            pltpu.VMEM((TPW, NL), jnp.float32),
            pltpu.VMEM((TPW, NL), jnp.float32),
            pltpu.SemaphoreType.DMA,
            pltpu.SemaphoreType.DMA,
            pltpu.SemaphoreType.DMA,
            pltpu.SemaphoreType.DMA,
        ],
    )


def _combine_body(ys_hbm, pa_hbm, pb_hbm, wa_hbm, wb_hbm, out_hbm,
                  a0, b0, a1, b1, obuf, ia, ib, wab, wbb, s0a, s0b, s1a, s1b):
    wid = lax.axis_index("s") * NC + lax.axis_index("c")
    base = wid * TPW
    pltpu.sync_copy(pa_hbm.at[pl.ds(base, TPW)], ia)
    pltpu.sync_copy(pb_hbm.at[pl.ds(base, TPW)], ib)
    pltpu.sync_copy(wa_hbm.at[pl.ds(base, TPW)], wab)
    pltpu.sync_copy(wb_hbm.at[pl.ds(base, TPW)], wbb)

    nch = TPW // CCH
    abufs, bbufs = (a0, a1), (b0, b1)
    sas, sbs = (s0a, s1a), (s0b, s1b)

    def start(c):
        p = c % 2
        iva = ia[pl.ds(c * CCH, CCH)]
        ivb = ib[pl.ds(c * CCH, CCH)]
        ca = pltpu.async_copy(ys_hbm.at[iva], abufs[p], sas[p])
        cb = pltpu.async_copy(ys_hbm.at[ivb], bbufs[p], sbs[p])
        return ca, cb

    pend = start(0)
    for c in range(nch):
        p = c % 2
        pend[0].wait()
        pend[1].wait()
        if c + 1 < nch:
            pend = start(c + 1)
        ab, bb = abufs[p], bbufs[p]

        def token_body(j, carry):
            wa = wab[c * CCH + j]
            wb = wbb[c * CCH + j]
            for h in range(H // NL):
                sl = pl.ds(h * NL, NL)
                obuf[j, sl] = ab[j, sl] * wa + bb[j, sl] * wb
            return carry

        lax.fori_loop(0, CCH, token_body, 0)
        pltpu.sync_copy(obuf, out_hbm.at[pl.ds(base + c * CCH, CCH)])


@jax.jit
def kernel(x, gate_w, expert_w):
    oh1, oh2, wa16, wb16, xb = _router(x, gate_w)
    pa, pb, bexp, bact = _positions(oh1, oh2)
    xb32 = lax.bitcast_convert_type(xb.reshape(T, H2, 2), jnp.int32)
    xs32 = _dispatch_kernel()(xb32, pa, pb)
    xs = lax.bitcast_convert_type(xs32, jnp.bfloat16).reshape(CAP, H)
    ys = _grouped_matmul(bexp, bact, xs, expert_w)
    return _combine_kernel()(ys, pa, pb, wa16, wb16)


# speedup vs baseline: 3.3279x; 3.3279x over previous
import functools

import jax
import jax.numpy as jnp
from jax import lax
from jax.experimental import pallas as pl
from jax.experimental.pallas import tpu as pltpu
from jax.experimental.pallas import tpu_sc as plsc

E = 8
H = 1024
H2 = H // 2
T = 4096
TOPK = 2
BM = 512
CAP = T * TOPK + E * BM
NB = CAP // BM

NC, NS, NL = 2, 16, 16
NW = NC * NS
TPW = T // NW
DCH = 64
CCH = 16


@functools.cache
def _mesh():
    return plsc.VectorSubcoreMesh(core_axis_name="c", subcore_axis_name="s")


def _router_body(x_ref, gw_ref, oh1_ref, oh2_ref, wa_ref, wb_ref):
    x = x_ref[...]
    logits = lax.dot_general(
        x, gw_ref[...], (((1,), (1,)), ((), ())),
        precision=lax.Precision.DEFAULT,
        preferred_element_type=jnp.float32)
    m = jnp.max(logits, axis=-1, keepdims=True)
    el = jnp.exp(logits - m)
    probs = el / jnp.sum(el, axis=-1, keepdims=True)

    eidx = lax.broadcasted_iota(jnp.int32, probs.shape, 1)
    m1 = jnp.max(probs, axis=-1, keepdims=True)
    i1 = jnp.min(jnp.where(probs == m1, eidx, E), axis=-1, keepdims=True)
    oh1 = (eidx == i1).astype(jnp.float32)
    probs2 = jnp.where(oh1 > 0, -1.0, probs)
    m2 = jnp.max(probs2, axis=-1, keepdims=True)
    i2 = jnp.min(jnp.where(probs2 == m2, eidx, E), axis=-1, keepdims=True)
    oh2 = (eidx == i2).astype(jnp.float32)

    oh1_ref[...] = oh1
    oh2_ref[...] = oh2
    wa_ref[...] = jnp.broadcast_to(m1, (x.shape[0], NL))
    wb_ref[...] = jnp.broadcast_to(m2, (x.shape[0], NL))


def _router(x, gate_w):
    return pl.pallas_call(
        _router_body,
        in_specs=[
            pl.BlockSpec((T, H), lambda: (0, 0)),
            pl.BlockSpec((E, H), lambda: (0, 0)),
        ],
        out_specs=[
            pl.BlockSpec((T, E), lambda: (0, 0)),
            pl.BlockSpec((T, E), lambda: (0, 0)),
            pl.BlockSpec((T, NL), lambda: (0, 0)),
            pl.BlockSpec((T, NL), lambda: (0, 0)),
        ],
        out_shape=[
            jax.ShapeDtypeStruct((T, E), jnp.float32),
            jax.ShapeDtypeStruct((T, E), jnp.float32),
            jax.ShapeDtypeStruct((T, NL), jnp.float32),
            jax.ShapeDtypeStruct((T, NL), jnp.float32),
        ],
    )(x, gate_w)


def _positions(oh1, oh2):
    onehot = jnp.concatenate([oh1, oh2], axis=0)
    counts = jnp.sum(onehot, axis=0)
    csum = jnp.cumsum(onehot, axis=0)
    rank = jnp.sum(csum * onehot, axis=1) - 1.0
    pc = jnp.ceil(counts / BM) * BM
    po = jnp.concatenate([jnp.zeros((1,), jnp.float32),
                          jnp.cumsum(pc)[:-1]])
    pos = (jnp.sum(onehot * po[None, :], axis=1) + rank).astype(jnp.int32)
    pa, pb = pos[:T], pos[T:]

    bstart = (jnp.arange(NB, dtype=jnp.float32) * BM)
    act = (bstart[:, None] >= po[None, :]) & (bstart[:, None] < (po + pc)[None, :])
    bexp = jnp.argmax(act, axis=1).astype(jnp.int32)
    bact = jnp.any(act, axis=1).astype(jnp.int32)
    return pa, pb, bexp, bact


@functools.cache
def _dispatch_kernel():
    return pl.kernel(
        _dispatch_body,
        out_type=jax.ShapeDtypeStruct((CAP, H), jnp.float32),
        mesh=_mesh(),
        scratch_types=[
            pltpu.VMEM((DCH, H), jnp.float32),
            pltpu.VMEM((DCH,), jnp.int32),
            pltpu.VMEM((DCH,), jnp.int32),
            pltpu.SemaphoreType.DMA,
            pltpu.SemaphoreType.DMA,
        ],
    )


def _dispatch_body(x_hbm, pa_hbm, pb_hbm, xs_hbm, xbuf, ia, ib, s1, s2):
    wid = lax.axis_index("s") * NC + lax.axis_index("c")
    for c in range(TPW // DCH):
        base = wid * TPW + c * DCH
        pltpu.sync_copy(x_hbm.at[pl.ds(base, DCH)], xbuf)
        pltpu.sync_copy(pa_hbm.at[pl.ds(base, DCH)], ia)
        pltpu.sync_copy(pb_hbm.at[pl.ds(base, DCH)], ib)
        ca = pltpu.async_copy(xbuf, xs_hbm.at[ia], s1)
        cb = pltpu.async_copy(xbuf, xs_hbm.at[ib], s2)
        ca.wait()
        cb.wait()


def _mm_body(be_ref, ba_ref, xs_ref, w_ref, o_ref):
    b = pl.program_id(0)

    @pl.when(ba_ref[b] == 1)
    def _():
        w = w_ref[pl.ds(be_ref[b], 1)]
        o_ref[...] = lax.dot_general(
            xs_ref[...], w[0], (((1,), (1,)), ((), ())),
            precision=lax.Precision.DEFAULT,
            preferred_element_type=jnp.float32)


def _grouped_matmul(bexp, bact, xs, expert_w):
    grid_spec = pltpu.PrefetchScalarGridSpec(
        num_scalar_prefetch=2,
        grid=(NB,),
        in_specs=[
            pl.BlockSpec((BM, H),
                         lambda b, be, ba: (jnp.where(ba[b] == 1, b, 0), 0)),
            pl.BlockSpec((E, H, H), lambda b, be, ba: (0, 0, 0)),
        ],
        out_specs=pl.BlockSpec(
            (BM, H), lambda b, be, ba: (jnp.where(ba[b] == 1, b, NB), 0)),
    )
    return pl.pallas_call(
        _mm_body,
        grid_spec=grid_spec,
        out_shape=jax.ShapeDtypeStruct((CAP + BM, H), jnp.float32),
    )(bexp, bact, xs, expert_w)


@functools.cache
def _combine_kernel():
    return pl.kernel(
        _combine_body,
        out_type=jax.ShapeDtypeStruct((T, H), jnp.float32),
        mesh=_mesh(),
        scratch_types=[
            pltpu.VMEM((CCH, H), jnp.float32),
            pltpu.VMEM((CCH, H), jnp.float32),
            pltpu.VMEM((CCH, H), jnp.float32),
            pltpu.VMEM((CCH, H), jnp.float32),
            pltpu.VMEM((CCH, H), jnp.float32),
            pltpu.VMEM((TPW,), jnp.int32),
            pltpu.VMEM((TPW,), jnp.int32),
            pltpu.VMEM((TPW, NL), jnp.float32),
            pltpu.VMEM((TPW, NL), jnp.float32),
            pltpu.SemaphoreType.DMA,
            pltpu.SemaphoreType.DMA,
            pltpu.SemaphoreType.DMA,
            pltpu.SemaphoreType.DMA,
        ],
    )


def _combine_body(ys_hbm, pa_hbm, pb_hbm, wa_hbm, wb_hbm, out_hbm,
                  a0, b0, a1, b1, obuf, ia, ib, wab, wbb, s0a, s0b, s1a, s1b):
    wid = lax.axis_index("s") * NC + lax.axis_index("c")
    base = wid * TPW
    pltpu.sync_copy(pa_hbm.at[pl.ds(base, TPW)], ia)
    pltpu.sync_copy(pb_hbm.at[pl.ds(base, TPW)], ib)
    pltpu.sync_copy(wa_hbm.at[pl.ds(base, TPW)], wab)
    pltpu.sync_copy(wb_hbm.at[pl.ds(base, TPW)], wbb)

    nch = TPW // CCH
    abufs, bbufs = (a0, a1), (b0, b1)
    sas, sbs = (s0a, s1a), (s0b, s1b)

    def start(c):
        p = c % 2
        iva = ia[pl.ds(c * CCH, CCH)]
        ivb = ib[pl.ds(c * CCH, CCH)]
        ca = pltpu.async_copy(ys_hbm.at[iva], abufs[p], sas[p])
        cb = pltpu.async_copy(ys_hbm.at[ivb], bbufs[p], sbs[p])
        return ca, cb

    pend = start(0)
    for c in range(nch):
        p = c % 2
        pend[0].wait()
        pend[1].wait()
        if c + 1 < nch:
            pend = start(c + 1)
        ab, bb = abufs[p], bbufs[p]

        def token_body(j, carry):
            wa = wab[c * CCH + j]
            wb = wbb[c * CCH + j]
            for h in range(H // NL):
                sl = pl.ds(h * NL, NL)
                obuf[j, sl] = ab[j, sl] * wa + bb[j, sl] * wb
            return carry

        lax.fori_loop(0, CCH, token_body, 0)
        pltpu.sync_copy(obuf, out_hbm.at[pl.ds(base + c * CCH, CCH)])


@jax.jit
def kernel(x, gate_w, expert_w):
    oh1, oh2, wa16, wb16 = _router(x, gate_w)
    pa, pb, bexp, bact = _positions(oh1, oh2)
    xs = _dispatch_kernel()(x, pa, pb)
    ys = _grouped_matmul(bexp, bact, xs, expert_w)
    return _combine_kernel()(ys, pa, pb, wa16, wb16)
